# Initial kernel scaffold; baseline (speedup 1.0000x reference)
import jax
import jax.numpy as jnp
from jax.experimental import pallas as pl


def kernel(x, edge_index, h, c, W1, b1, W2, b2, W_ih, W_hh, b_ih, b_hh):
    raise NotImplementedError("write your pallas kernel here")


# speedup vs baseline: 9.0283x; 9.0283x over previous
import functools

import jax
import jax.numpy as jnp
from jax import lax
from jax.experimental import pallas as pl
from jax.experimental.pallas import tpu as pltpu, tpu_sc as plsc

N = 10000
E = 320000
D = 128
DH = 128

NC = 2
NS = 16
NW = NC * NS
CH = 128
CHUNKS = 80
E_PAD = NW * CHUNKS * CH
NP = 10240
ROWS_PER_TILE = NP // NS
DUMP_STEPS = ROWS_PER_TILE // CH

_mesh = plsc.VectorSubcoreMesh(core_axis_name="c", subcore_axis_name="s")


@functools.partial(
    pl.kernel,
    out_type=[
        jax.ShapeDtypeStruct((NP,), jnp.float32),
        jax.ShapeDtypeStruct((NP,), jnp.float32),
    ],
    mesh=_mesh,
    scratch_types=[
        pltpu.VMEM((CHUNKS, CH), jnp.int32),
        pltpu.VMEM((NP,), jnp.float32),
        pltpu.VMEM((NS, ROWS_PER_TILE), jnp.float32),
        pltpu.VMEM((ROWS_PER_TILE,), jnp.float32),
        pltpu.VMEM_SHARED((NS, NP), jnp.float32),
    ],
    compiler_params=pltpu.CompilerParams(needs_layout_passes=False),
)
def _deg_kernel(dst_hbm, out0_hbm, out1_hbm,
                didx_v, degl_v, buf_v, red_v, stage_sh):
    c = lax.axis_index("c")
    s = lax.axis_index("s")
    wid = s * NC + c

    pltpu.sync_copy(dst_hbm.at[wid], didx_v)

    one16 = jnp.ones((16,), jnp.float32)
    zero16 = jnp.zeros((16,), jnp.float32)

    def fill_zero(i, carry):
        degl_v[pl.ds(i * 16, 16)] = zero16
        return carry

    lax.fori_loop(0, NP // 16, fill_zero, 0)

    def chunk_body(j, carry):
        for k in range(CH // 16):
            d = didx_v[j, pl.ds(k * 16, 16)]
            plsc.addupdate_scatter(degl_v, [d], one16)
        return carry

    lax.fori_loop(0, CHUNKS, chunk_body, 0)

    pltpu.sync_copy(degl_v, stage_sh.at[s])
    plsc.subcore_barrier()

    pltpu.sync_copy(stage_sh.at[:, pl.ds(s * ROWS_PER_TILE, ROWS_PER_TILE)], buf_v)

    def red_body(k, carry):
        acc = buf_v[0, pl.ds(k * 16, 16)]
        for r in range(1, NS):
            acc = acc + buf_v[r, pl.ds(k * 16, 16)]
        red_v[pl.ds(k * 16, 16)] = acc
        return carry

    lax.fori_loop(0, ROWS_PER_TILE // 16, red_body, 0)

    sl = pl.ds(s * ROWS_PER_TILE, ROWS_PER_TILE)

    @pl.when(c == 0)
    def _dump0():
        pltpu.sync_copy(red_v, out0_hbm.at[sl])

    @pl.when(c == 1)
    def _dump1():
        pltpu.sync_copy(red_v, out1_hbm.at[sl])


@functools.partial(
    pl.kernel,
    out_type=[
        jax.ShapeDtypeStruct((NP, D), jnp.float32),
        jax.ShapeDtypeStruct((NP, D), jnp.float32),
    ],
    mesh=_mesh,
    scratch_types=[
        pltpu.VMEM((CHUNKS, CH), jnp.int32),
        pltpu.VMEM((CHUNKS, CH), jnp.int32),
        pltpu.VMEM((CH, D), jnp.float32),
        pltpu.VMEM_SHARED((NP, D), jnp.float32),
        pltpu.SemaphoreType.DMA,
    ],
)
def _agg_kernel(src_hbm, dst_hbm, xs_hbm, out0_hbm, out1_hbm,
                sidx_v, didx_v, rows_v, acc_sh, sem):
    c = lax.axis_index("c")
    s = lax.axis_index("s")
    wid = s * NC + c

    pltpu.sync_copy(src_hbm.at[wid], sidx_v)
    pltpu.sync_copy(dst_hbm.at[wid], didx_v)

    zero16 = jnp.zeros((16,), jnp.float32)

    def fill_zero(i, carry):
        for k in range(D // 16):
            rows_v[i, pl.ds(k * 16, 16)] = zero16
        return carry

    lax.fori_loop(0, CH, fill_zero, 0)

    for k in range(DUMP_STEPS):
        pltpu.sync_copy(rows_v, acc_sh.at[pl.ds(s * ROWS_PER_TILE + k * CH, CH)])
    plsc.subcore_barrier()

    def body(j, carry):
        pltpu.async_copy(xs_hbm.at[sidx_v.at[j]], rows_v, sem).wait()
        pltpu.sync_copy(rows_v, acc_sh.at[didx_v.at[j]], add=True)
        return carry

    lax.fori_loop(0, CHUNKS, body, 0)
    plsc.subcore_barrier()

    sl = pl.ds(s * ROWS_PER_TILE, ROWS_PER_TILE)

    @pl.when(c == 0)
    def _dump0():
        pltpu.sync_copy(acc_sh.at[sl], out0_hbm.at[sl])

    @pl.when(c == 1)
    def _dump1():
        pltpu.sync_copy(acc_sh.at[sl], out1_hbm.at[sl])


def _dinv_body(d0_ref, d1_ref, dinv_ref):
    d = d0_ref[...] + d1_ref[...] + 1.0
    dinv_ref[...] = lax.rsqrt(d)


def _xs1_body(x_ref, w_ref, dinv_ref, out_ref):
    xw = jnp.dot(x_ref[...], w_ref[...], preferred_element_type=jnp.float32)
    out_ref[...] = xw * dinv_ref[...]


def _mid_body(p0_ref, p1_ref, xs_ref, dinv_ref, b1_ref, w2_ref, out_ref):
    agg = p0_ref[...] + p1_ref[...] + xs_ref[...]
    z1 = jax.nn.relu(dinv_ref[...] * agg + b1_ref[...])
    zw = jnp.dot(z1, w2_ref[...], preferred_element_type=jnp.float32)
    out_ref[...] = zw * dinv_ref[...]


def _lstm_body(q0_ref, q1_ref, xs_ref, dinv_ref, b2_ref,
               h_ref, c_ref, wih_ref, whh_ref, bias_ref, h2_ref, c2_ref):
    agg = q0_ref[...] + q1_ref[...] + xs_ref[...]
    z2 = dinv_ref[...] * agg + b2_ref[...]
    gates = (jnp.dot(z2, wih_ref[...], preferred_element_type=jnp.float32)
             + jnp.dot(h_ref[...], whh_ref[...], preferred_element_type=jnp.float32)
             + bias_ref[...])
    ig = jax.nn.sigmoid(gates[:, 0:DH])
    fg = jax.nn.sigmoid(gates[:, DH:2 * DH])
    gg = jnp.tanh(gates[:, 2 * DH:3 * DH])
    og = jax.nn.sigmoid(gates[:, 3 * DH:4 * DH])
    c2 = fg * c_ref[...] + ig * gg
    h2_ref[...] = og * jnp.tanh(c2)
    c2_ref[...] = c2


_BLK = 1000
_GRID = N // _BLK


def _rowspec(shape):
    return pl.BlockSpec(shape, lambda i: (i, 0))


def _fullspec(shape):
    nd = len(shape)
    return pl.BlockSpec(shape, lambda i, _n=nd: (0,) * _n)


def kernel(x, edge_index, h, c, W1, b1, W2, b2, W_ih, W_hh, b_ih, b_hh):
    src = edge_index[0]
    dst = edge_index[1]
    pad = E_PAD - E
    src_p = jnp.concatenate([src, jnp.zeros((pad,), jnp.int32)])
    dst_p = jnp.concatenate([dst, jnp.full((pad,), N, jnp.int32)])
    src3 = src_p.reshape(NW, CHUNKS, CH)
    dst3 = dst_p.reshape(NW, CHUNKS, CH)

    d0, d1 = _deg_kernel(dst3)
    d0r = d0.reshape(NP, 1)
    d1r = d1.reshape(NP, 1)

    dinv = pl.pallas_call(
        _dinv_body,
        grid=(1,),
        in_specs=[_fullspec((NP, 1)), _fullspec((NP, 1))],
        out_specs=_fullspec((NP, 1)),
        out_shape=jax.ShapeDtypeStruct((NP, 1), jnp.float32),
    )(d0r, d1r)

    xs1 = pl.pallas_call(
        _xs1_body,
        grid=(_GRID,),
        in_specs=[
            _rowspec((_BLK, D)),
            _fullspec((D, DH)),
            _rowspec((_BLK, 1)),
        ],
        out_specs=_rowspec((_BLK, DH)),
        out_shape=jax.ShapeDtypeStruct((N, DH), jnp.float32),
    )(x, W1, dinv)

    p0, p1 = _agg_kernel(src3, dst3, xs1)

    b1r = b1.reshape(1, DH)
    xs2 = pl.pallas_call(
        _mid_body,
        grid=(_GRID,),
        in_specs=[
            _rowspec((_BLK, DH)),
            _rowspec((_BLK, DH)),
            _rowspec((_BLK, DH)),
            _rowspec((_BLK, 1)),
            _fullspec((1, DH)),
            _fullspec((DH, DH)),
        ],
        out_specs=_rowspec((_BLK, DH)),
        out_shape=jax.ShapeDtypeStruct((N, DH), jnp.float32),
    )(p0, p1, xs1, dinv, b1r, W2)

    q0, q1 = _agg_kernel(src3, dst3, xs2)

    b2r = b2.reshape(1, DH)
    wihT = W_ih.T
    whhT = W_hh.T
    bias = (b_ih + b_hh).reshape(1, 4 * DH)
    h2, c2 = pl.pallas_call(
        _lstm_body,
        grid=(_GRID,),
        in_specs=[
            _rowspec((_BLK, DH)),
            _rowspec((_BLK, DH)),
            _rowspec((_BLK, DH)),
            _rowspec((_BLK, 1)),
            _fullspec((1, DH)),
            _rowspec((_BLK, DH)),
            _rowspec((_BLK, DH)),
            _fullspec((DH, 4 * DH)),
            _fullspec((DH, 4 * DH)),
            _fullspec((1, 4 * DH)),
        ],
        out_specs=[_rowspec((_BLK, DH)), _rowspec((_BLK, DH))],
        out_shape=[
            jax.ShapeDtypeStruct((N, DH), jnp.float32),
            jax.ShapeDtypeStruct((N, DH), jnp.float32),
        ],
    )(q0, q1, xs2, dinv, b2r, h, c, wihT, whhT, bias)

    return h2, c2


# speedup vs baseline: 9.0455x; 1.0019x over previous
import functools

import jax
import jax.numpy as jnp
from jax import lax
from jax.experimental import pallas as pl
from jax.experimental.pallas import tpu as pltpu, tpu_sc as plsc

N = 10000
E = 320000
D = 128
DH = 128

NC = 2
NS = 16
NW = NC * NS
CH = 128
CHUNKS = 80
E_PAD = NW * CHUNKS * CH
NP = 10240
ROWS_PER_TILE = NP // NS
DUMP_STEPS = ROWS_PER_TILE // CH

_mesh = plsc.VectorSubcoreMesh(core_axis_name="c", subcore_axis_name="s")


@functools.partial(
    pl.kernel,
    out_type=[
        jax.ShapeDtypeStruct((NP,), jnp.float32),
        jax.ShapeDtypeStruct((NP,), jnp.float32),
    ],
    mesh=_mesh,
    scratch_types=[
        pltpu.VMEM((CHUNKS, CH), jnp.int32),
        pltpu.VMEM((NP,), jnp.float32),
        pltpu.VMEM((NS, ROWS_PER_TILE), jnp.float32),
        pltpu.VMEM((ROWS_PER_TILE,), jnp.float32),
        pltpu.VMEM_SHARED((NS, NP), jnp.float32),
    ],
    compiler_params=pltpu.CompilerParams(needs_layout_passes=False),
)
def _deg_kernel(dst_hbm, out0_hbm, out1_hbm,
                didx_v, degl_v, buf_v, red_v, stage_sh):
    c = lax.axis_index("c")
    s = lax.axis_index("s")
    wid = s * NC + c

    pltpu.sync_copy(dst_hbm.at[wid], didx_v)

    one16 = jnp.ones((16,), jnp.float32)
    zero16 = jnp.zeros((16,), jnp.float32)

    def fill_zero(i, carry):
        degl_v[pl.ds(i * 16, 16)] = zero16
        return carry

    lax.fori_loop(0, NP // 16, fill_zero, 0)

    def chunk_body(j, carry):
        for k in range(CH // 16):
            d = didx_v[j, pl.ds(k * 16, 16)]
            plsc.addupdate_scatter(degl_v, [d], one16)
        return carry

    lax.fori_loop(0, CHUNKS, chunk_body, 0)

    pltpu.sync_copy(degl_v, stage_sh.at[s])
    plsc.subcore_barrier()

    pltpu.sync_copy(stage_sh.at[:, pl.ds(s * ROWS_PER_TILE, ROWS_PER_TILE)], buf_v)

    def red_body(k, carry):
        acc = buf_v[0, pl.ds(k * 16, 16)]
        for r in range(1, NS):
            acc = acc + buf_v[r, pl.ds(k * 16, 16)]
        red_v[pl.ds(k * 16, 16)] = acc
        return carry

    lax.fori_loop(0, ROWS_PER_TILE // 16, red_body, 0)

    sl = pl.ds(s * ROWS_PER_TILE, ROWS_PER_TILE)

    @pl.when(c == 0)
    def _dump0():
        pltpu.sync_copy(red_v, out0_hbm.at[sl])

    @pl.when(c == 1)
    def _dump1():
        pltpu.sync_copy(red_v, out1_hbm.at[sl])


@functools.partial(
    pl.kernel,
    out_type=[
        jax.ShapeDtypeStruct((NP, D), jnp.float32),
        jax.ShapeDtypeStruct((NP, D), jnp.float32),
    ],
    mesh=_mesh,
    scratch_types=[
        pltpu.VMEM((CHUNKS, CH), jnp.int32),
        pltpu.VMEM((CHUNKS, CH), jnp.int32),
        pltpu.VMEM((CH, D), jnp.float32),
        pltpu.VMEM_SHARED((NP, D), jnp.float32),
        pltpu.SemaphoreType.DMA,
    ],
)
def _agg_kernel(src_hbm, dst_hbm, xs_hbm, out0_hbm, out1_hbm,
                sidx_v, didx_v, rows_v, acc_sh, sem):
    c = lax.axis_index("c")
    s = lax.axis_index("s")
    wid = s * NC + c

    pltpu.sync_copy(src_hbm.at[wid], sidx_v)
    pltpu.sync_copy(dst_hbm.at[wid], didx_v)

    zero16 = jnp.zeros((16,), jnp.float32)

    def fill_zero(i, carry):
        for k in range(D // 16):
            rows_v[i, pl.ds(k * 16, 16)] = zero16
        return carry

    lax.fori_loop(0, CH, fill_zero, 0)

    for k in range(DUMP_STEPS):
        pltpu.sync_copy(rows_v, acc_sh.at[pl.ds(s * ROWS_PER_TILE + k * CH, CH)])
    plsc.subcore_barrier()

    def body(j, carry):
        pltpu.async_copy(xs_hbm.at[sidx_v.at[j]], rows_v, sem).wait()
        pltpu.sync_copy(rows_v, acc_sh.at[didx_v.at[j]], add=True)
        return carry

    lax.fori_loop(0, CHUNKS, body, 0)
    plsc.subcore_barrier()

    sl = pl.ds(s * ROWS_PER_TILE, ROWS_PER_TILE)

    @pl.when(c == 0)
    def _dump0():
        pltpu.sync_copy(acc_sh.at[sl], out0_hbm.at[sl])

    @pl.when(c == 1)
    def _dump1():
        pltpu.sync_copy(acc_sh.at[sl], out1_hbm.at[sl])


def _dinv_body(d0_ref, d1_ref, dinv_ref):
    d = d0_ref[...] + d1_ref[...] + 1.0
    dinv_ref[...] = lax.rsqrt(d)


def _xs1_body(x_ref, w_ref, dinv_ref, out_ref):
    xw = jnp.dot(x_ref[...], w_ref[...], preferred_element_type=jnp.float32)
    out_ref[...] = xw * dinv_ref[...]


def _mid_body(p0_ref, p1_ref, xs_ref, dinv_ref, b1_ref, w2_ref, out_ref):
    agg = p0_ref[...] + p1_ref[...] + xs_ref[...]
    z1 = jax.nn.relu(dinv_ref[...] * agg + b1_ref[...])
    zw = jnp.dot(z1, w2_ref[...], preferred_element_type=jnp.float32)
    out_ref[...] = zw * dinv_ref[...]


def _lstm_body(q0_ref, q1_ref, xs_ref, dinv_ref, b2_ref,
               h_ref, c_ref, wih_ref, whh_ref, bias_ref, h2_ref, c2_ref):
    agg = q0_ref[...] + q1_ref[...] + xs_ref[...]
    z2 = dinv_ref[...] * agg + b2_ref[...]
    gates = (jnp.dot(z2, wih_ref[...], preferred_element_type=jnp.float32)
             + jnp.dot(h_ref[...], whh_ref[...], preferred_element_type=jnp.float32)
             + bias_ref[...])
    ig = jax.nn.sigmoid(gates[:, 0:DH])
    fg = jax.nn.sigmoid(gates[:, DH:2 * DH])
    gg = jnp.tanh(gates[:, 2 * DH:3 * DH])
    og = jax.nn.sigmoid(gates[:, 3 * DH:4 * DH])
    c2 = fg * c_ref[...] + ig * gg
    h2_ref[...] = og * jnp.tanh(c2)
    c2_ref[...] = c2


_BLK = 1000
_GRID = N // _BLK


def _rowspec(shape):
    return pl.BlockSpec(shape, lambda i: (i, 0))


def _fullspec(shape):
    nd = len(shape)
    return pl.BlockSpec(shape, lambda i, _n=nd: (0,) * _n)


def kernel(x, edge_index, h, c, W1, b1, W2, b2, W_ih, W_hh, b_ih, b_hh):
    src = edge_index[0]
    dst = edge_index[1]
    pad = E_PAD - E
    src_p = jnp.concatenate([src, jnp.zeros((pad,), jnp.int32)])
    pad_dst = N + jnp.arange(pad, dtype=jnp.int32) % (NP - N)
    dst_p = jnp.concatenate([dst, pad_dst])
    src3 = src_p.reshape(NW, CHUNKS, CH)
    dst3 = dst_p.reshape(NW, CHUNKS, CH)

    d0, d1 = _deg_kernel(dst3)
    d0r = d0.reshape(NP, 1)
    d1r = d1.reshape(NP, 1)

    dinv = pl.pallas_call(
        _dinv_body,
        grid=(1,),
        in_specs=[_fullspec((NP, 1)), _fullspec((NP, 1))],
        out_specs=_fullspec((NP, 1)),
        out_shape=jax.ShapeDtypeStruct((NP, 1), jnp.float32),
    )(d0r, d1r)

    xs1 = pl.pallas_call(
        _xs1_body,
        grid=(_GRID,),
        in_specs=[
            _rowspec((_BLK, D)),
            _fullspec((D, DH)),
            _rowspec((_BLK, 1)),
        ],
        out_specs=_rowspec((_BLK, DH)),
        out_shape=jax.ShapeDtypeStruct((N, DH), jnp.float32),
    )(x, W1, dinv)

    p0, p1 = _agg_kernel(src3, dst3, xs1)

    b1r = b1.reshape(1, DH)
    xs2 = pl.pallas_call(
        _mid_body,
        grid=(_GRID,),
        in_specs=[
            _rowspec((_BLK, DH)),
            _rowspec((_BLK, DH)),
            _rowspec((_BLK, DH)),
            _rowspec((_BLK, 1)),
            _fullspec((1, DH)),
            _fullspec((DH, DH)),
        ],
        out_specs=_rowspec((_BLK, DH)),
        out_shape=jax.ShapeDtypeStruct((N, DH), jnp.float32),
    )(p0, p1, xs1, dinv, b1r, W2)

    q0, q1 = _agg_kernel(src3, dst3, xs2)

    b2r = b2.reshape(1, DH)
    wihT = W_ih.T
    whhT = W_hh.T
    bias = (b_ih + b_hh).reshape(1, 4 * DH)
    h2, c2 = pl.pallas_call(
        _lstm_body,
        grid=(_GRID,),
        in_specs=[
            _rowspec((_BLK, DH)),
            _rowspec((_BLK, DH)),
            _rowspec((_BLK, DH)),
            _rowspec((_BLK, 1)),
            _fullspec((1, DH)),
            _rowspec((_BLK, DH)),
            _rowspec((_BLK, DH)),
            _fullspec((DH, 4 * DH)),
            _fullspec((DH, 4 * DH)),
            _fullspec((1, 4 * DH)),
        ],
        out_specs=[_rowspec((_BLK, DH)), _rowspec((_BLK, DH))],
        out_shape=[
            jax.ShapeDtypeStruct((N, DH), jnp.float32),
            jax.ShapeDtypeStruct((N, DH), jnp.float32),
        ],
    )(q0, q1, xs2, dinv, b2r, h, c, wihT, whhT, bias)

    return h2, c2


# speedup vs baseline: 9.7368x; 1.0764x over previous
import functools

import jax
import jax.numpy as jnp
from jax import lax
from jax.experimental import pallas as pl
from jax.experimental.pallas import tpu as pltpu, tpu_sc as plsc

N = 10000
E = 320000
D = 128
DH = 128

NC = 2
NS = 16
NW = NC * NS
CH = 128
CHUNKS = 80
E_PAD = NW * CHUNKS * CH
NP = 10240
ROWS_PER_TILE = NP // NS

_mesh = plsc.VectorSubcoreMesh(core_axis_name="c", subcore_axis_name="s")


@functools.partial(
    pl.kernel,
    out_type=[
        jax.ShapeDtypeStruct((NP,), jnp.float32),
        jax.ShapeDtypeStruct((NP,), jnp.float32),
    ],
    mesh=_mesh,
    scratch_types=[
        pltpu.VMEM((CHUNKS, CH), jnp.int32),
        pltpu.VMEM((NP,), jnp.float32),
        pltpu.VMEM((NS, ROWS_PER_TILE), jnp.float32),
        pltpu.VMEM((ROWS_PER_TILE,), jnp.float32),
        pltpu.VMEM_SHARED((NS, NP), jnp.float32),
    ],
    compiler_params=pltpu.CompilerParams(needs_layout_passes=False),
)
def _deg_kernel(key_hbm, out0_hbm, out1_hbm,
                kidx_v, degl_v, buf_v, red_v, stage_sh):
    c = lax.axis_index("c")
    s = lax.axis_index("s")
    wid = s * NC + c

    pltpu.sync_copy(key_hbm.at[wid], kidx_v)

    one16 = jnp.ones((16,), jnp.float32)
    zero16 = jnp.zeros((16,), jnp.float32)

    def fill_zero(i, carry):
        degl_v[pl.ds(i * 16, 16)] = zero16
        return carry

    lax.fori_loop(0, NP // 16, fill_zero, 0)

    def chunk_body(j, carry):
        for k in range(CH // 16):
            kv = kidx_v[j, pl.ds(k * 16, 16)]
            d = lax.shift_right_logical(kv, 14)
            plsc.addupdate_scatter(degl_v, [d], one16)
        return carry

    lax.fori_loop(0, CHUNKS, chunk_body, 0)

    pltpu.sync_copy(degl_v, stage_sh.at[s])
    plsc.subcore_barrier()

    pltpu.sync_copy(stage_sh.at[:, pl.ds(s * ROWS_PER_TILE, ROWS_PER_TILE)], buf_v)

    def red_body(k, carry):
        acc = buf_v[0, pl.ds(k * 16, 16)]
        for r in range(1, NS):
            acc = acc + buf_v[r, pl.ds(k * 16, 16)]
        red_v[pl.ds(k * 16, 16)] = acc
        return carry

    lax.fori_loop(0, ROWS_PER_TILE // 16, red_body, 0)

    sl = pl.ds(s * ROWS_PER_TILE, ROWS_PER_TILE)

    @pl.when(c == 0)
    def _dump0():
        pltpu.sync_copy(red_v, out0_hbm.at[sl])

    @pl.when(c == 1)
    def _dump1():
        pltpu.sync_copy(red_v, out1_hbm.at[sl])


@functools.partial(
    pl.kernel,
    out_type=[
        jax.ShapeDtypeStruct((NP, D), jnp.float32),
        jax.ShapeDtypeStruct((NP, D), jnp.float32),
    ],
    mesh=_mesh,
    scratch_types=[
        pltpu.VMEM((CHUNKS, CH), jnp.int32),
        pltpu.VMEM((2, CH), jnp.int32),
        pltpu.VMEM((2, CH), jnp.int32),
        pltpu.VMEM((CH, D), jnp.float32),
        pltpu.VMEM((CH, D), jnp.float32),
        pltpu.VMEM_SHARED((NP, D), jnp.float32),
        pltpu.SemaphoreType.DMA,
        pltpu.SemaphoreType.DMA,
    ],
    compiler_params=pltpu.CompilerParams(needs_layout_passes=False),
)
def _agg_kernel(key_hbm, xs_hbm, out0_hbm, out1_hbm,
                kidx_v, sidx_s, didx_s, rows_v, rows2_v, acc_sh, sem, sem2):
    c = lax.axis_index("c")
    s = lax.axis_index("s")
    wid = s * NC + c

    pltpu.sync_copy(key_hbm.at[wid], kidx_v)

    zero16 = jnp.zeros((16,), jnp.float32)

    def fill_zero(i, carry):
        for k in range(D // 16):
            rows_v[i, pl.ds(k * 16, 16)] = zero16
        return carry

    lax.fori_loop(0, CH, fill_zero, 0)

    base = s * ROWS_PER_TILE
    nfull = ROWS_PER_TILE // CH
    for k in range(nfull):
        pltpu.sync_copy(rows_v, acc_sh.at[pl.ds(base + k * CH, CH)])
    rem = ROWS_PER_TILE - nfull * CH
    if rem:
        pltpu.sync_copy(rows_v.at[pl.ds(0, rem)],
                        acc_sh.at[pl.ds(base + nfull * CH, rem)])
    plsc.subcore_barrier()

    def unpack(j, b):
        for k in range(CH // 16):
            kv = kidx_v[j, pl.ds(k * 16, 16)]
            sidx_s[b, pl.ds(k * 16, 16)] = kv & 0x3FFF
            didx_s[b, pl.ds(k * 16, 16)] = lax.shift_right_logical(kv, 14)

    def gather(b, buf, s_):
        return pltpu.async_copy(xs_hbm.at[sidx_s.at[b]], buf, s_)

    def gwait(b, buf, s_):
        pltpu.make_async_copy(xs_hbm.at[sidx_s.at[b]], buf, s_).wait()

    def scatter(b, buf):
        pltpu.sync_copy(buf, acc_sh.at[didx_s.at[b]], add=True)

    unpack(0, 0)
    gather(0, rows_v, sem)

    def body(i, carry):
        j = 2 * i
        unpack(j + 1, 1)
        gather(1, rows2_v, sem2)
        gwait(0, rows_v, sem)
        scatter(0, rows_v)
        unpack(j + 2, 0)
        gather(0, rows_v, sem)
        gwait(1, rows2_v, sem2)
        scatter(1, rows2_v)
        return carry

    lax.fori_loop(0, (CHUNKS - 2) // 2, body, 0)

    unpack(CHUNKS - 1, 1)
    gather(1, rows2_v, sem2)
    gwait(0, rows_v, sem)
    scatter(0, rows_v)
    gwait(1, rows2_v, sem2)
    scatter(1, rows2_v)
    plsc.subcore_barrier()

    sl = pl.ds(base, ROWS_PER_TILE)

    @pl.when(c == 0)
    def _dump0():
        pltpu.sync_copy(acc_sh.at[sl], out0_hbm.at[sl])

    @pl.when(c == 1)
    def _dump1():
        pltpu.sync_copy(acc_sh.at[sl], out1_hbm.at[sl])


def _dinv_body(d0_ref, d1_ref, dinv_ref):
    d = d0_ref[...] + d1_ref[...] + 1.0
    dinv_ref[...] = lax.rsqrt(d)


def _xs1_body(x_ref, w_ref, dinv_ref, out_ref):
    xw = jnp.dot(x_ref[...], w_ref[...], preferred_element_type=jnp.float32)
    out_ref[...] = xw * dinv_ref[...]


def _mid_body(p0_ref, p1_ref, xs_ref, dinv_ref, b1_ref, w2_ref, out_ref):
    agg = p0_ref[...] + p1_ref[...] + xs_ref[...]
    z1 = jax.nn.relu(dinv_ref[...] * agg + b1_ref[...])
    zw = jnp.dot(z1, w2_ref[...], preferred_element_type=jnp.float32)
    out_ref[...] = zw * dinv_ref[...]


def _lstm_body(q0_ref, q1_ref, xs_ref, dinv_ref, b2_ref,
               h_ref, c_ref, wih_ref, whh_ref, bias_ref, h2_ref, c2_ref):
    agg = q0_ref[...] + q1_ref[...] + xs_ref[...]
    z2 = dinv_ref[...] * agg + b2_ref[...]
    gates = (jnp.dot(z2, wih_ref[...], preferred_element_type=jnp.float32)
             + jnp.dot(h_ref[...], whh_ref[...], preferred_element_type=jnp.float32)
             + bias_ref[...])
    ig = jax.nn.sigmoid(gates[:, 0:DH])
    fg = jax.nn.sigmoid(gates[:, DH:2 * DH])
    gg = jnp.tanh(gates[:, 2 * DH:3 * DH])
    og = jax.nn.sigmoid(gates[:, 3 * DH:4 * DH])
    c2 = fg * c_ref[...] + ig * gg
    h2_ref[...] = og * jnp.tanh(c2)
    c2_ref[...] = c2


_BLK = 1000
_GRID = N // _BLK


def _rowspec(shape):
    return pl.BlockSpec(shape, lambda i: (i, 0))


def _fullspec(shape):
    nd = len(shape)
    return pl.BlockSpec(shape, lambda i, _n=nd: (0,) * _n)


def kernel(x, edge_index, h, c, W1, b1, W2, b2, W_ih, W_hh, b_ih, b_hh):
    src = edge_index[0]
    dst = edge_index[1]
    pad = E_PAD - E
    src_p = jnp.concatenate([src, jnp.zeros((pad,), jnp.int32)])
    pad_dst = N + jnp.arange(pad, dtype=jnp.int32) % (NP - N)
    dst_p = jnp.concatenate([dst, pad_dst])
    key3 = (src_p | (dst_p << 14)).reshape(NW, CHUNKS, CH)

    d0, d1 = _deg_kernel(key3)
    d0r = d0.reshape(NP, 1)
    d1r = d1.reshape(NP, 1)

    dinv = pl.pallas_call(
        _dinv_body,
        grid=(1,),
        in_specs=[_fullspec((NP, 1)), _fullspec((NP, 1))],
        out_specs=_fullspec((NP, 1)),
        out_shape=jax.ShapeDtypeStruct((NP, 1), jnp.float32),
    )(d0r, d1r)

    xs1 = pl.pallas_call(
        _xs1_body,
        grid=(_GRID,),
        in_specs=[
            _rowspec((_BLK, D)),
            _fullspec((D, DH)),
            _rowspec((_BLK, 1)),
        ],
        out_specs=_rowspec((_BLK, DH)),
        out_shape=jax.ShapeDtypeStruct((N, DH), jnp.float32),
    )(x, W1, dinv)

    p0, p1 = _agg_kernel(key3, xs1)

    b1r = b1.reshape(1, DH)
    xs2 = pl.pallas_call(
        _mid_body,
        grid=(_GRID,),
        in_specs=[
            _rowspec((_BLK, DH)),
            _rowspec((_BLK, DH)),
            _rowspec((_BLK, DH)),
            _rowspec((_BLK, 1)),
            _fullspec((1, DH)),
            _fullspec((DH, DH)),
        ],
        out_specs=_rowspec((_BLK, DH)),
        out_shape=jax.ShapeDtypeStruct((N, DH), jnp.float32),
    )(p0, p1, xs1, dinv, b1r, W2)

    q0, q1 = _agg_kernel(key3, xs2)

    b2r = b2.reshape(1, DH)
    wihT = W_ih.T
    whhT = W_hh.T
    bias = (b_ih + b_hh).reshape(1, 4 * DH)
    h2, c2 = pl.pallas_call(
        _lstm_body,
        grid=(_GRID,),
        in_specs=[
            _rowspec((_BLK, DH)),
            _rowspec((_BLK, DH)),
            _rowspec((_BLK, DH)),
            _rowspec((_BLK, 1)),
            _fullspec((1, DH)),
            _rowspec((_BLK, DH)),
            _rowspec((_BLK, DH)),
            _fullspec((DH, 4 * DH)),
            _fullspec((DH, 4 * DH)),
            _fullspec((1, 4 * DH)),
        ],
        out_specs=[_rowspec((_BLK, DH)), _rowspec((_BLK, DH))],
        out_shape=[
            jax.ShapeDtypeStruct((N, DH), jnp.float32),
            jax.ShapeDtypeStruct((N, DH), jnp.float32),
        ],
    )(q0, q1, xs2, dinv, b2r, h, c, wihT, whhT, bias)

    return h2, c2
